# baseline (speedup 1.0000x reference)
import functools

import jax
import jax.numpy as jnp
from jax import lax
from jax.experimental import pallas as pl
from jax.experimental.pallas import tpu as pltpu
from jax.experimental.pallas import tpu_sc as plsc

N = 10000
E = 320000
F = 128
G = 64
MF = 32
EMB = 256
HID = 256
OUT = 128
NR = 32
CUT = 6.0
H = EMB // 2

NSUB = 16
CHUNK = 64
M_TEC = ((E // NSUB + CHUNK - 1) // CHUNK) * CHUNK
E_PAD = M_TEC * NSUB
NCHUNK = M_TEC // CHUNK
E_IDX = E_PAD + 2 * CHUNK
W_EDGE = E // 32

NB = 1000
NBLK = N // NB
EB = 2048
EBLK = E_PAD // EB

_MESH = plsc.VectorSubcoreMesh(core_axis_name="c", subcore_axis_name="s")
_SC_PARAMS = pltpu.CompilerParams(needs_layout_passes=False)


@functools.partial(
    pl.kernel,
    mesh=_MESH,
    compiler_params=_SC_PARAMS,
    out_type=jax.ShapeDtypeStruct((E,), jnp.float32),
    scratch_types=[
        pltpu.VMEM((N,), jnp.float32),
        pltpu.VMEM((N,), jnp.float32),
        pltpu.VMEM((N,), jnp.float32),
        pltpu.VMEM((W_EDGE,), jnp.int32),
        pltpu.VMEM((W_EDGE,), jnp.int32),
        pltpu.VMEM((W_EDGE,), jnp.float32),
    ],
)
def _sc_dist2(posx, posy, posz, src, dst, d2, px, py, pz, sb, db, ob):
    c = lax.axis_index("c")
    s = lax.axis_index("s")
    w = c * NSUB + s
    base = w * W_EDGE
    pltpu.sync_copy(posx, px)
    pltpu.sync_copy(posy, py)
    pltpu.sync_copy(posz, pz)
    pltpu.sync_copy(src.at[pl.ds(base, W_EDGE)], sb)
    pltpu.sync_copy(dst.at[pl.ds(base, W_EDGE)], db)

    def body(i, _):
        o = i * 16
        sv = sb[pl.ds(o, 16)]
        dv = db[pl.ds(o, 16)]
        dx = plsc.load_gather(px, [sv]) - plsc.load_gather(px, [dv])
        dy = plsc.load_gather(py, [sv]) - plsc.load_gather(py, [dv])
        dz = plsc.load_gather(pz, [sv]) - plsc.load_gather(pz, [dv])
        ob[pl.ds(o, 16)] = dx * dx + dy * dy + dz * dz
        return 0

    lax.fori_loop(0, W_EDGE // 16, body, 0)
    pltpu.sync_copy(ob, d2.at[pl.ds(base, W_EDGE)])


@functools.partial(
    pl.kernel,
    mesh=_MESH,
    compiler_params=_SC_PARAMS,
    out_type=[jax.ShapeDtypeStruct((N, H), jnp.float32)] * 2,
    scratch_types=[
        pltpu.VMEM((CHUNK,), jnp.int32),
        pltpu.VMEM((CHUNK,), jnp.int32),
        pltpu.VMEM((1, CHUNK), jnp.int32),
        pltpu.VMEM((1, CHUNK), jnp.int32),
        pltpu.VMEM((1, CHUNK), jnp.int32),
        pltpu.VMEM((1, CHUNK), jnp.int32),
        pltpu.VMEM((CHUNK, H), jnp.float32),
        pltpu.VMEM((CHUNK, H), jnp.float32),
        pltpu.VMEM((CHUNK, H), jnp.float32),
        pltpu.VMEM((CHUNK, H), jnp.float32),
        pltpu.VMEM_SHARED((N, H), jnp.float32),
        pltpu.SemaphoreType.DMA,
        pltpu.SemaphoreType.DMA,
        pltpu.SemaphoreType.DMA,
        pltpu.SemaphoreType.DMA,
        pltpu.SemaphoreType.DMA,
        pltpu.SemaphoreType.DMA,
        pltpu.SemaphoreType.DMA,
        pltpu.SemaphoreType.DMA,
    ],
)
def _sc_msg(h_lo, h_hi, ef_lo, ef_hi, sd, agg_lo, agg_hi,
            ib0, ib1, sx0, sx1, dx0, dx1, hb0, hb1, eb0, eb1, accum,
            gs0, gs1, es0, es1, ss0, ss1, is0, is1):
    c = lax.axis_index("c")
    s = lax.axis_index("s")
    zeros16 = jnp.zeros((16,), jnp.float32)
    r0 = s * 624
    nit = 39 + (s == 15).astype(jnp.int32)
    ib = (ib0, ib1)
    sx = (sx0, sx1)
    dx = (dx0, dx1)
    hb = (hb0, hb1)
    eb = (eb0, eb1)
    gs = (gs0, gs1)
    es = (es0, es1)
    ss = (ss0, ss1)
    isem = (is0, is1)

    def run_half(h_t, ef_t, out_t):
        def zrow(r, _):
            for k in range(H // 16):
                hb0[r, pl.ds(k * 16, 16)] = zeros16
            return 0

        lax.fori_loop(0, 16, zrow, 0)

        def zcp(t, _):
            pltpu.sync_copy(hb0.at[pl.ds(0, 16)],
                            accum.at[pl.ds(pl.multiple_of(r0 + t * 16, 8),
                                           16)])
            return 0

        lax.fori_loop(0, nit, zcp, 0)
        plsc.subcore_barrier()

        def idx_fetch(j, b):
            off = pl.multiple_of(s * M_TEC + j * CHUNK, 8)
            pltpu.async_copy(sd.at[pl.ds(off, CHUNK)], ib[b], isem[b])

        def issue(j, b):
            off = pl.multiple_of(s * M_TEC + j * CHUNK, 8)
            pltpu.make_async_copy(sd.at[pl.ds(off, CHUNK)], ib[b],
                                  isem[b]).wait()

            def dec(k, _):
                sl = pl.ds(k * 16, 16)
                v = ib[b][sl]
                sx[b][0, sl] = v & 0xFFFF
                dx[b][0, sl] = lax.shift_right_logical(v, 16)
                return 0

            lax.fori_loop(0, CHUNK // 16, dec, 0, unroll=4)
            idx_fetch(j + 2, b)
            pltpu.async_copy(h_t.at[sx[b].at[0]], hb[b], gs[b])
            pltpu.async_copy(ef_t.at[pl.ds(off, CHUNK)], eb[b], es[b])

        def wait_scatter(b):
            pltpu.make_async_copy(hb[b], accum.at[dx[b].at[0]], ss[b]).wait()

        def compute(j, b):
            off = pl.multiple_of(s * M_TEC + j * CHUNK, 8)
            pltpu.make_async_copy(h_t.at[sx[b].at[0]], hb[b], gs[b]).wait()
            pltpu.make_async_copy(ef_t.at[pl.ds(off, CHUNK)], eb[b],
                                  es[b]).wait()

            def mrow(r, _):
                for k in range(H // 16):
                    sl = pl.ds(k * 16, 16)
                    hb[b][r, sl] = hb[b][r, sl] * eb[b][r, sl]
                return 0

            lax.fori_loop(0, CHUNK, mrow, 0, unroll=2)
            pltpu.async_copy(hb[b], accum.at[dx[b].at[0]], ss[b], add=True)

        idx_fetch(0, 0)
        idx_fetch(1, 1)
        issue(0, 0)
        issue(1, 1)

        def pair(t, _):
            j = 2 * t
            compute(j, 0)
            wait_scatter(0)
            issue(j + 2, 0)
            compute(j + 1, 1)
            wait_scatter(1)
            issue(j + 3, 1)
            return 0

        lax.fori_loop(0, (NCHUNK - 3) // 2, pair, 0)
        compute(NCHUNK - 3, 0)
        wait_scatter(0)
        issue(NCHUNK - 1, 0)
        compute(NCHUNK - 2, 1)
        compute(NCHUNK - 1, 0)
        wait_scatter(1)
        wait_scatter(0)
        off0 = pl.multiple_of(s * M_TEC + NCHUNK * CHUNK, 8)
        pltpu.make_async_copy(sd.at[pl.ds(off0, CHUNK)], ib[1],
                              isem[1]).wait()
        off1 = pl.multiple_of(s * M_TEC + (NCHUNK + 1) * CHUNK, 8)
        pltpu.make_async_copy(sd.at[pl.ds(off1, CHUNK)], ib[0],
                              isem[0]).wait()
        plsc.subcore_barrier()

        def wcp(t, _):
            rt = pl.multiple_of(r0 + t * 16, 8)
            pltpu.sync_copy(accum.at[pl.ds(rt, 16)], hb0.at[pl.ds(0, 16)])
            pltpu.sync_copy(hb0.at[pl.ds(0, 16)], out_t.at[pl.ds(rt, 16)])
            return 0

        lax.fori_loop(0, nit, wcp, 0)

    @pl.when(c == 0)
    def _():
        run_half(h_lo, ef_lo, agg_lo)

    @pl.when(c == 1)
    def _():
        run_half(h_hi, ef_hi, agg_hi)


def _gelu(x):
    return jax.nn.gelu(x)


def _emb_body(x_ref, mi_ref, ai_ref, b_ref, molx_ref, W1_ref, wmi_ref,
              wai_ref, Wm_ref, bemb_ref, lo_ref, hi_ref):
    molW = jnp.dot(molx_ref[...], Wm_ref[...],
                   preferred_element_type=jnp.float32)
    oh = (b_ref[...] == lax.broadcasted_iota(jnp.int32, (1, G), 1)
          ).astype(jnp.float32)
    h = jnp.dot(x_ref[...], W1_ref[...], preferred_element_type=jnp.float32)
    h += mi_ref[...] * wmi_ref[...]
    h += ai_ref[...] * wai_ref[...]
    h += jnp.dot(oh, molW, preferred_element_type=jnp.float32)
    h += bemb_ref[...]
    h = _gelu(h)
    lo_ref[...] = h[:, :H]
    hi_ref[...] = h[:, H:]


def _ef_body(d2_ref, Wr_ref, elo, ehi):
    i = pl.program_id(0)
    d2 = d2_ref[...]
    d = jnp.sqrt(d2 + 1e-8)
    env = 0.5 * (jnp.cos(jnp.pi * jnp.clip(d / CUT, 0.0, 1.0)) + 1.0)
    centers = lax.broadcasted_iota(jnp.int32, (1, NR), 1).astype(
        jnp.float32) * (CUT / (NR - 1))
    gamma = (NR / CUT) ** 2
    rbf = jnp.exp(-gamma * (d - centers) ** 2) * env
    gidx = i * EB + lax.broadcasted_iota(jnp.int32, (EB, 1), 0)
    rbf = jnp.where(gidx < E, rbf, 0.0)
    ef = jnp.dot(rbf, Wr_ref[...], preferred_element_type=jnp.float32)
    elo[...] = ef[:, :H]
    ehi[...] = ef[:, H:]


def _upd_body(hlo_ref, hhi_ref, alo_ref, ahi_ref, W_ref, b_ref,
              lo_ref, hi_ref):
    hc = jnp.concatenate([hlo_ref[...] + alo_ref[...],
                          hhi_ref[...] + ahi_ref[...]], axis=1)
    h = _gelu(jnp.dot(hc, W_ref[...], preferred_element_type=jnp.float32)
              + b_ref[...])
    lo_ref[...] = h[:, :H]
    hi_ref[...] = h[:, H:]


def _upd2_body(hlo_ref, hhi_ref, alo_ref, ahi_ref, W_ref, b_ref, t_ref,
               lo_ref, hi_ref, mx_ref):
    hc = jnp.concatenate([hlo_ref[...] + alo_ref[...],
                          hhi_ref[...] + ahi_ref[...]], axis=1)
    h = _gelu(jnp.dot(hc, W_ref[...], preferred_element_type=jnp.float32)
              + b_ref[...])
    lo_ref[...] = h[:, :H]
    hi_ref[...] = h[:, H:]
    gate = h * t_ref[...]
    m = jnp.max(gate, axis=0, keepdims=True)

    @pl.when(pl.program_id(0) == 0)
    def _():
        mx_ref[...] = jnp.full((8, EMB), -jnp.inf, jnp.float32)

    mx_ref[...] = jnp.maximum(mx_ref[...], jnp.broadcast_to(m, (8, EMB)))


def _pool_body(hlo_ref, hhi_ref, b_ref, mx_ref, t_ref, bp_ref, molx_ref,
               Wmol_ref, bmol_ref, Wf1_ref, bf1_ref, Wf2_ref, bf2_ref,
               Wo_ref, out_ref, Z_ref, S_ref):
    i = pl.program_id(0)
    h = jnp.concatenate([hlo_ref[...], hhi_ref[...]], axis=1)
    gate = h * t_ref[...]
    a = jnp.exp(gate - mx_ref[0:1, :])
    oh = (b_ref[...] == lax.broadcasted_iota(jnp.int32, (1, G), 1)
          ).astype(jnp.float32)
    dn = (((0,), (0,)), ((), ()))
    z = lax.dot_general(oh, a, dn, preferred_element_type=jnp.float32)
    sm = lax.dot_general(oh, a * (h + bp_ref[...]), dn,
                         preferred_element_type=jnp.float32)

    @pl.when(i == 0)
    def _():
        Z_ref[...] = jnp.zeros((G, EMB), jnp.float32)
        S_ref[...] = jnp.zeros((G, EMB), jnp.float32)

    Z_ref[...] += z
    S_ref[...] += sm

    @pl.when(i == NBLK - 1)
    def _():
        g = S_ref[...] / (Z_ref[...] + 1e-12)
        mol = jnp.dot(molx_ref[...], Wmol_ref[...],
                      preferred_element_type=jnp.float32) + bmol_ref[...]
        gc = jnp.concatenate([g, mol], axis=1)
        g1 = _gelu(jnp.dot(gc, Wf1_ref[...],
                           preferred_element_type=jnp.float32) + bf1_ref[...])
        g2 = _gelu(jnp.dot(g1, Wf2_ref[...],
                           preferred_element_type=jnp.float32) + bf2_ref[...])
        out_ref[...] = jnp.dot(g2, Wo_ref[...],
                               preferred_element_type=jnp.float32)


def _full(shape):
    return pl.BlockSpec(shape, lambda i: tuple(0 for _ in shape))


def _nq(i):
    return (i, 0)


def kernel(x, pos, batch, ptr, mol_x, num_graphs, aux_ind, mol_ind,
           edge_index, W_emb, b_emb, W_rbf1, W_upd1, b_upd1, W_rbf2, W_upd2,
           b_upd2, t_pool, b_pool, W_mol, b_mol, W_fc1, b_fc1, W_fc2, b_fc2,
           W_out):
    f32 = jnp.float32
    src = edge_index[0]
    dst = edge_index[1]
    sd = jnp.concatenate(
        [src + (dst << 16), jnp.zeros((E_IDX - E,), jnp.int32)])
    batch1 = batch.reshape(N, 1)
    mi1 = mol_ind.reshape(N, 1)
    ai1 = aux_ind.reshape(N, 1)

    d2 = _sc_dist2(pos[:, 0], pos[:, 1], pos[:, 2], src, dst)
    d2p = jnp.concatenate([d2, jnp.zeros((E_PAD - E,), f32)]).reshape(E_PAD, 1)

    W1 = W_emb[:F]
    wmi = W_emb[F:F + 1]
    wai = W_emb[F + 1:F + 2]
    Wm = W_emb[F + 2:]
    h_lo, h_hi = pl.pallas_call(
        _emb_body,
        grid=(NBLK,),
        in_specs=[
            pl.BlockSpec((NB, F), _nq),
            pl.BlockSpec((NB, 1), _nq),
            pl.BlockSpec((NB, 1), _nq),
            pl.BlockSpec((NB, 1), _nq),
            _full((G, MF)),
            _full((F, EMB)),
            _full((1, EMB)),
            _full((1, EMB)),
            _full((MF, EMB)),
            _full((1, EMB)),
        ],
        out_specs=[pl.BlockSpec((NB, H), _nq)] * 2,
        out_shape=[jax.ShapeDtypeStruct((N, H), f32)] * 2,
    )(x, mi1, ai1, batch1, mol_x, W1, wmi, wai, Wm, b_emb.reshape(1, EMB))

    nebl = E_PAD // EB + 1
    ef_call = pl.pallas_call(
        _ef_body,
        grid=(nebl,),
        in_specs=[
            pl.BlockSpec((EB, 1), _nq),
            _full((NR, EMB)),
        ],
        out_specs=[pl.BlockSpec((EB, H), _nq)] * 2,
        out_shape=[jax.ShapeDtypeStruct((E_PAD, H), f32)] * 2,
    )
    e1lo, e1hi = ef_call(d2p, W_rbf1)
    e2lo, e2hi = ef_call(d2p, W_rbf2)

    upd_call = pl.pallas_call(
        _upd_body,
        grid=(NBLK,),
        in_specs=[pl.BlockSpec((NB, H), _nq)] * 4
        + [_full((EMB, EMB)), _full((1, EMB))],
        out_specs=[pl.BlockSpec((NB, H), _nq)] * 2,
        out_shape=[jax.ShapeDtypeStruct((N, H), f32)] * 2,
    )
    upd2_call = pl.pallas_call(
        _upd2_body,
        grid=(NBLK,),
        in_specs=[pl.BlockSpec((NB, H), _nq)] * 4
        + [_full((EMB, EMB)), _full((1, EMB)), _full((1, EMB))],
        out_specs=[pl.BlockSpec((NB, H), _nq)] * 2 + [_full((8, EMB))],
        out_shape=[jax.ShapeDtypeStruct((N, H), f32)] * 2
        + [jax.ShapeDtypeStruct((8, EMB), f32)],
    )

    a_lo, a_hi = _sc_msg(h_lo, h_hi, e1lo, e1hi, sd)
    h_lo, h_hi = upd_call(h_lo, h_hi, a_lo, a_hi, W_upd1,
                          b_upd1.reshape(1, EMB))

    a_lo, a_hi = _sc_msg(h_lo, h_hi, e2lo, e2hi, sd)
    h_lo, h_hi, gmax = upd2_call(h_lo, h_hi, a_lo, a_hi, W_upd2,
                                 b_upd2.reshape(1, EMB),
                                 t_pool.reshape(1, EMB))

    out = pl.pallas_call(
        _pool_body,
        grid=(NBLK,),
        in_specs=[pl.BlockSpec((NB, H), _nq)] * 2
        + [
            pl.BlockSpec((NB, 1), _nq),
            _full((8, EMB)),
            _full((1, EMB)),
            _full((1, EMB)),
            _full((G, MF)),
            _full((MF, MF)),
            _full((1, MF)),
            _full((EMB + MF, HID)),
            _full((1, HID)),
            _full((HID, HID)),
            _full((1, HID)),
            _full((HID, OUT)),
        ],
        out_specs=_full((G, OUT)),
        out_shape=jax.ShapeDtypeStruct((G, OUT), f32),
        scratch_shapes=[pltpu.VMEM((G, EMB), f32), pltpu.VMEM((G, EMB), f32)],
    )(h_lo, h_hi, batch1, gmax, t_pool.reshape(1, EMB),
      b_pool.reshape(1, EMB), mol_x, W_mol, b_mol.reshape(1, MF), W_fc1,
      b_fc1.reshape(1, HID), W_fc2, b_fc2.reshape(1, HID), W_out)
    return out


# speedup vs baseline: 1.5738x; 1.5738x over previous
import functools

import jax
import jax.numpy as jnp
from jax import lax
from jax.experimental import pallas as pl
from jax.experimental.pallas import tpu as pltpu
from jax.experimental.pallas import tpu_sc as plsc

N = 10000
E = 320000
F = 128
G = 64
MF = 32
EMB = 256
HID = 256
OUT = 128
NR = 32
CUT = 6.0
H = EMB // 2

NSUB = 16
CHUNK = 64
M_TEC = ((E // NSUB + CHUNK - 1) // CHUNK) * CHUNK
E_PAD = M_TEC * NSUB
NCHUNK = M_TEC // CHUNK
E_IDX = E_PAD + 2 * CHUNK
W_EDGE = E // 32

NB = 1000
NBLK = N // NB
EB = 2048

_MESH = plsc.VectorSubcoreMesh(core_axis_name="c", subcore_axis_name="s")
_SC_PARAMS = pltpu.CompilerParams(needs_layout_passes=False)


@functools.partial(
    pl.kernel,
    mesh=_MESH,
    compiler_params=_SC_PARAMS,
    out_type=jax.ShapeDtypeStruct((E,), jnp.float32),
    scratch_types=[
        pltpu.VMEM((N,), jnp.float32),
        pltpu.VMEM((N,), jnp.float32),
        pltpu.VMEM((N,), jnp.float32),
        pltpu.VMEM((W_EDGE,), jnp.int32),
        pltpu.VMEM((W_EDGE,), jnp.int32),
        pltpu.VMEM((W_EDGE,), jnp.float32),
    ],
)
def _sc_dist2(posx, posy, posz, src, dst, d2, px, py, pz, sb, db, ob):
    c = lax.axis_index("c")
    s = lax.axis_index("s")
    w = c * NSUB + s
    base = w * W_EDGE
    pltpu.sync_copy(posx, px)
    pltpu.sync_copy(posy, py)
    pltpu.sync_copy(posz, pz)
    pltpu.sync_copy(src.at[pl.ds(base, W_EDGE)], sb)
    pltpu.sync_copy(dst.at[pl.ds(base, W_EDGE)], db)

    def body(i, _):
        o = i * 16
        sv = sb[pl.ds(o, 16)]
        dv = db[pl.ds(o, 16)]
        dx = plsc.load_gather(px, [sv]) - plsc.load_gather(px, [dv])
        dy = plsc.load_gather(py, [sv]) - plsc.load_gather(py, [dv])
        dz = plsc.load_gather(pz, [sv]) - plsc.load_gather(pz, [dv])
        ob[pl.ds(o, 16)] = dx * dx + dy * dy + dz * dz
        return 0

    lax.fori_loop(0, W_EDGE // 16, body, 0)
    pltpu.sync_copy(ob, d2.at[pl.ds(base, W_EDGE)])


@functools.partial(
    pl.kernel,
    mesh=_MESH,
    compiler_params=_SC_PARAMS,
    out_type=[jax.ShapeDtypeStruct((N, H), jnp.float32)] * 2,
    scratch_types=[
        pltpu.VMEM((CHUNK,), jnp.int32),
        pltpu.VMEM((CHUNK,), jnp.int32),
        pltpu.VMEM((1, CHUNK), jnp.int32),
        pltpu.VMEM((1, CHUNK), jnp.int32),
        pltpu.VMEM((1, CHUNK), jnp.int32),
        pltpu.VMEM((1, CHUNK), jnp.int32),
        pltpu.VMEM((CHUNK, H), jnp.float32),
        pltpu.VMEM((CHUNK, H), jnp.float32),
        pltpu.VMEM((CHUNK, H), jnp.float32),
        pltpu.VMEM((CHUNK, H), jnp.float32),
        pltpu.VMEM_SHARED((N, H), jnp.float32),
        pltpu.SemaphoreType.DMA,
        pltpu.SemaphoreType.DMA,
        pltpu.SemaphoreType.DMA,
        pltpu.SemaphoreType.DMA,
        pltpu.SemaphoreType.DMA,
        pltpu.SemaphoreType.DMA,
        pltpu.SemaphoreType.DMA,
        pltpu.SemaphoreType.DMA,
    ],
)
def _sc_msg(h_lo, h_hi, ef_lo, ef_hi, sd, agg_lo, agg_hi,
            ib0, ib1, sx0, sx1, dx0, dx1, hb0, hb1, eb0, eb1, accum,
            gs0, gs1, es0, es1, ss0, ss1, is0, is1):
    c = lax.axis_index("c")
    s = lax.axis_index("s")
    zeros16 = jnp.zeros((16,), jnp.float32)
    r0 = s * 624
    nit = 39 + (s == 15).astype(jnp.int32)
    ib = (ib0, ib1)
    sx = (sx0, sx1)
    dx = (dx0, dx1)
    hb = (hb0, hb1)
    eb = (eb0, eb1)
    gs = (gs0, gs1)
    es = (es0, es1)
    ss = (ss0, ss1)
    isem = (is0, is1)

    def run_half(h_t, ef_t, out_t):
        def zrow(r, _):
            for k in range(H // 16):
                hb0[r, pl.ds(k * 16, 16)] = zeros16
            return 0

        lax.fori_loop(0, 16, zrow, 0)

        def zcp(t, _):
            pltpu.sync_copy(hb0.at[pl.ds(0, 16)],
                            accum.at[pl.ds(pl.multiple_of(r0 + t * 16, 8),
                                           16)])
            return 0

        lax.fori_loop(0, nit, zcp, 0)
        plsc.subcore_barrier()

        def idx_fetch(j, b):
            off = pl.multiple_of(s * M_TEC + j * CHUNK, 8)
            pltpu.async_copy(sd.at[pl.ds(off, CHUNK)], ib[b], isem[b])

        def issue(j, b):
            off = pl.multiple_of(s * M_TEC + j * CHUNK, 8)
            pltpu.make_async_copy(sd.at[pl.ds(off, CHUNK)], ib[b],
                                  isem[b]).wait()

            def dec(k, _):
                sl = pl.ds(k * 16, 16)
                v = ib[b][sl]
                sx[b][0, sl] = v & 0xFFFF
                dx[b][0, sl] = lax.shift_right_logical(v, 16)
                return 0

            lax.fori_loop(0, CHUNK // 16, dec, 0, unroll=4)
            idx_fetch(j + 2, b)
            pltpu.async_copy(h_t.at[sx[b].at[0]], hb[b], gs[b])
            pltpu.async_copy(ef_t.at[pl.ds(off, CHUNK)], eb[b], es[b])

        def wait_scatter(b):
            pltpu.make_async_copy(hb[b], accum.at[dx[b].at[0]], ss[b]).wait()

        def compute(j, b):
            off = pl.multiple_of(s * M_TEC + j * CHUNK, 8)
            pltpu.make_async_copy(h_t.at[sx[b].at[0]], hb[b], gs[b]).wait()
            pltpu.make_async_copy(ef_t.at[pl.ds(off, CHUNK)], eb[b],
                                  es[b]).wait()

            def mrow(r4, _):
                for q in range(4):
                    r = r4 * 4 + q
                    for k in range(H // 16):
                        sl = pl.ds(k * 16, 16)
                        hb[b][r, sl] = hb[b][r, sl] * eb[b][r, sl]
                return 0

            lax.fori_loop(0, CHUNK // 4, mrow, 0)
            pltpu.async_copy(hb[b], accum.at[dx[b].at[0]], ss[b], add=True)

        idx_fetch(0, 0)
        idx_fetch(1, 1)
        issue(0, 0)
        issue(1, 1)

        def pair(t, _):
            j = 2 * t
            compute(j, 0)
            wait_scatter(0)
            issue(j + 2, 0)
            compute(j + 1, 1)
            wait_scatter(1)
            issue(j + 3, 1)
            return 0

        lax.fori_loop(0, (NCHUNK - 3) // 2, pair, 0)
        compute(NCHUNK - 3, 0)
        wait_scatter(0)
        issue(NCHUNK - 1, 0)
        compute(NCHUNK - 2, 1)
        compute(NCHUNK - 1, 0)
        wait_scatter(1)
        wait_scatter(0)
        off0 = pl.multiple_of(s * M_TEC + NCHUNK * CHUNK, 8)
        pltpu.make_async_copy(sd.at[pl.ds(off0, CHUNK)], ib[1],
                              isem[1]).wait()
        off1 = pl.multiple_of(s * M_TEC + (NCHUNK + 1) * CHUNK, 8)
        pltpu.make_async_copy(sd.at[pl.ds(off1, CHUNK)], ib[0],
                              isem[0]).wait()
        plsc.subcore_barrier()

        def wcp(t, _):
            rt = pl.multiple_of(r0 + t * 16, 8)
            pltpu.sync_copy(accum.at[pl.ds(rt, 16)], hb0.at[pl.ds(0, 16)])
            pltpu.sync_copy(hb0.at[pl.ds(0, 16)], out_t.at[pl.ds(rt, 16)])
            return 0

        lax.fori_loop(0, nit, wcp, 0)

    @pl.when(c == 0)
    def _():
        run_half(h_lo, ef_lo, agg_lo)

    @pl.when(c == 1)
    def _():
        run_half(h_hi, ef_hi, agg_hi)


def _gelu(x):
    return jax.nn.gelu(x)


def _emb_body(x_ref, mi_ref, ai_ref, b_ref, molx_ref, W1_ref, wmi_ref,
              wai_ref, Wm_ref, bemb_ref, lo_ref, hi_ref):
    molW = jnp.dot(molx_ref[...], Wm_ref[...],
                   preferred_element_type=jnp.float32)
    oh = (b_ref[...] == lax.broadcasted_iota(jnp.int32, (1, G), 1)
          ).astype(jnp.float32)
    h = jnp.dot(x_ref[...], W1_ref[...], preferred_element_type=jnp.float32)
    h += mi_ref[...] * wmi_ref[...]
    h += ai_ref[...] * wai_ref[...]
    h += jnp.dot(oh, molW, preferred_element_type=jnp.float32)
    h += bemb_ref[...]
    h = _gelu(h)
    lo_ref[...] = h[:, :H]
    hi_ref[...] = h[:, H:]


def _ef_body(d2_ref, Wr1_ref, Wr2_ref, e1lo, e1hi, e2lo, e2hi):
    i = pl.program_id(0)
    d2 = d2_ref[...]
    d = jnp.sqrt(d2 + 1e-8)
    env = 0.5 * (jnp.cos(jnp.pi * jnp.clip(d / CUT, 0.0, 1.0)) + 1.0)
    centers = lax.broadcasted_iota(jnp.int32, (1, NR), 1).astype(
        jnp.float32) * (CUT / (NR - 1))
    gamma = (NR / CUT) ** 2
    rbf = jnp.exp(-gamma * (d - centers) ** 2) * env
    gidx = i * EB + lax.broadcasted_iota(jnp.int32, (EB, 1), 0)
    rbf = jnp.where(gidx < E, rbf, 0.0)
    ef1 = jnp.dot(rbf, Wr1_ref[...], preferred_element_type=jnp.float32)
    ef2 = jnp.dot(rbf, Wr2_ref[...], preferred_element_type=jnp.float32)
    e1lo[...] = ef1[:, :H]
    e1hi[...] = ef1[:, H:]
    e2lo[...] = ef2[:, :H]
    e2hi[...] = ef2[:, H:]


def _upd_body(hlo_ref, hhi_ref, alo_ref, ahi_ref, W_ref, b_ref,
              lo_ref, hi_ref):
    hc = jnp.concatenate([hlo_ref[...] + alo_ref[...],
                          hhi_ref[...] + ahi_ref[...]], axis=1)
    h = _gelu(jnp.dot(hc, W_ref[...], preferred_element_type=jnp.float32)
              + b_ref[...])
    lo_ref[...] = h[:, :H]
    hi_ref[...] = h[:, H:]


def _upd2_body(hlo_ref, hhi_ref, alo_ref, ahi_ref, W_ref, b_ref, t_ref,
               lo_ref, hi_ref, mx_ref):
    hc = jnp.concatenate([hlo_ref[...] + alo_ref[...],
                          hhi_ref[...] + ahi_ref[...]], axis=1)
    h = _gelu(jnp.dot(hc, W_ref[...], preferred_element_type=jnp.float32)
              + b_ref[...])
    lo_ref[...] = h[:, :H]
    hi_ref[...] = h[:, H:]
    gate = h * t_ref[...]
    m = jnp.max(gate, axis=0, keepdims=True)

    @pl.when(pl.program_id(0) == 0)
    def _():
        mx_ref[...] = jnp.full((8, EMB), -jnp.inf, jnp.float32)

    mx_ref[...] = jnp.maximum(mx_ref[...], jnp.broadcast_to(m, (8, EMB)))


def _pool_body(hlo_ref, hhi_ref, b_ref, mx_ref, t_ref, bp_ref, molx_ref,
               Wmol_ref, bmol_ref, Wf1_ref, bf1_ref, Wf2_ref, bf2_ref,
               Wo_ref, out_ref, Z_ref, S_ref):
    i = pl.program_id(0)
    h = jnp.concatenate([hlo_ref[...], hhi_ref[...]], axis=1)
    gate = h * t_ref[...]
    a = jnp.exp(gate - mx_ref[0:1, :])
    oh = (b_ref[...] == lax.broadcasted_iota(jnp.int32, (1, G), 1)
          ).astype(jnp.float32)
    dn = (((0,), (0,)), ((), ()))
    z = lax.dot_general(oh, a, dn, preferred_element_type=jnp.float32)
    sm = lax.dot_general(oh, a * (h + bp_ref[...]), dn,
                         preferred_element_type=jnp.float32)

    @pl.when(i == 0)
    def _():
        Z_ref[...] = jnp.zeros((G, EMB), jnp.float32)
        S_ref[...] = jnp.zeros((G, EMB), jnp.float32)

    Z_ref[...] += z
    S_ref[...] += sm

    @pl.when(i == NBLK - 1)
    def _():
        g = S_ref[...] / (Z_ref[...] + 1e-12)
        mol = jnp.dot(molx_ref[...], Wmol_ref[...],
                      preferred_element_type=jnp.float32) + bmol_ref[...]
        gc = jnp.concatenate([g, mol], axis=1)
        g1 = _gelu(jnp.dot(gc, Wf1_ref[...],
                           preferred_element_type=jnp.float32) + bf1_ref[...])
        g2 = _gelu(jnp.dot(g1, Wf2_ref[...],
                           preferred_element_type=jnp.float32) + bf2_ref[...])
        out_ref[...] = jnp.dot(g2, Wo_ref[...],
                               preferred_element_type=jnp.float32)


def _full(shape):
    return pl.BlockSpec(shape, lambda i: tuple(0 for _ in shape))


def _nq(i):
    return (i, 0)


def kernel(x, pos, batch, ptr, mol_x, num_graphs, aux_ind, mol_ind,
           edge_index, W_emb, b_emb, W_rbf1, W_upd1, b_upd1, W_rbf2, W_upd2,
           b_upd2, t_pool, b_pool, W_mol, b_mol, W_fc1, b_fc1, W_fc2, b_fc2,
           W_out):
    f32 = jnp.float32
    src = edge_index[0]
    dst = edge_index[1]
    sd = jnp.concatenate(
        [src + (dst << 16), jnp.zeros((E_IDX - E,), jnp.int32)])
    batch1 = batch.reshape(N, 1)
    mi1 = mol_ind.reshape(N, 1)
    ai1 = aux_ind.reshape(N, 1)

    d2 = _sc_dist2(pos[:, 0], pos[:, 1], pos[:, 2], src, dst)
    d2p = jnp.concatenate([d2, jnp.zeros((E_PAD - E,), f32)]).reshape(E_PAD, 1)

    W1 = W_emb[:F]
    wmi = W_emb[F:F + 1]
    wai = W_emb[F + 1:F + 2]
    Wm = W_emb[F + 2:]
    h_lo, h_hi = pl.pallas_call(
        _emb_body,
        grid=(NBLK,),
        in_specs=[
            pl.BlockSpec((NB, F), _nq),
            pl.BlockSpec((NB, 1), _nq),
            pl.BlockSpec((NB, 1), _nq),
            pl.BlockSpec((NB, 1), _nq),
            _full((G, MF)),
            _full((F, EMB)),
            _full((1, EMB)),
            _full((1, EMB)),
            _full((MF, EMB)),
            _full((1, EMB)),
        ],
        out_specs=[pl.BlockSpec((NB, H), _nq)] * 2,
        out_shape=[jax.ShapeDtypeStruct((N, H), f32)] * 2,
    )(x, mi1, ai1, batch1, mol_x, W1, wmi, wai, Wm, b_emb.reshape(1, EMB))

    nebl = E_PAD // EB + 1
    e1lo, e1hi, e2lo, e2hi = pl.pallas_call(
        _ef_body,
        grid=(nebl,),
        in_specs=[
            pl.BlockSpec((EB, 1), _nq),
            _full((NR, EMB)),
            _full((NR, EMB)),
        ],
        out_specs=[pl.BlockSpec((EB, H), _nq)] * 4,
        out_shape=[jax.ShapeDtypeStruct((E_PAD, H), f32)] * 4,
    )(d2p, W_rbf1, W_rbf2)

    upd_call = pl.pallas_call(
        _upd_body,
        grid=(NBLK,),
        in_specs=[pl.BlockSpec((NB, H), _nq)] * 4
        + [_full((EMB, EMB)), _full((1, EMB))],
        out_specs=[pl.BlockSpec((NB, H), _nq)] * 2,
        out_shape=[jax.ShapeDtypeStruct((N, H), f32)] * 2,
    )
    upd2_call = pl.pallas_call(
        _upd2_body,
        grid=(NBLK,),
        in_specs=[pl.BlockSpec((NB, H), _nq)] * 4
        + [_full((EMB, EMB)), _full((1, EMB)), _full((1, EMB))],
        out_specs=[pl.BlockSpec((NB, H), _nq)] * 2 + [_full((8, EMB))],
        out_shape=[jax.ShapeDtypeStruct((N, H), f32)] * 2
        + [jax.ShapeDtypeStruct((8, EMB), f32)],
    )

    a_lo, a_hi = _sc_msg(h_lo, h_hi, e1lo, e1hi, sd)
    h_lo, h_hi = upd_call(h_lo, h_hi, a_lo, a_hi, W_upd1,
                          b_upd1.reshape(1, EMB))

    a_lo, a_hi = _sc_msg(h_lo, h_hi, e2lo, e2hi, sd)
    h_lo, h_hi, gmax = upd2_call(h_lo, h_hi, a_lo, a_hi, W_upd2,
                                 b_upd2.reshape(1, EMB),
                                 t_pool.reshape(1, EMB))

    out = pl.pallas_call(
        _pool_body,
        grid=(NBLK,),
        in_specs=[pl.BlockSpec((NB, H), _nq)] * 2
        + [
            pl.BlockSpec((NB, 1), _nq),
            _full((8, EMB)),
            _full((1, EMB)),
            _full((1, EMB)),
            _full((G, MF)),
            _full((MF, MF)),
            _full((1, MF)),
            _full((EMB + MF, HID)),
            _full((1, HID)),
            _full((HID, HID)),
            _full((1, HID)),
            _full((HID, OUT)),
        ],
        out_specs=_full((G, OUT)),
        out_shape=jax.ShapeDtypeStruct((G, OUT), f32),
        scratch_shapes=[pltpu.VMEM((G, EMB), f32), pltpu.VMEM((G, EMB), f32)],
    )(h_lo, h_hi, batch1, gmax, t_pool.reshape(1, EMB),
      b_pool.reshape(1, EMB), mol_x, W_mol, b_mol.reshape(1, MF), W_fc1,
      b_fc1.reshape(1, HID), W_fc2, b_fc2.reshape(1, HID), W_out)
    return out
